# baseline (speedup 1.0000x reference)
import functools

import jax
import jax.numpy as jnp
from jax import lax
from jax.experimental import pallas as pl
from jax.experimental.pallas import tpu as pltpu
from jax.experimental.pallas import tpu_sc as plsc

VOCAB = 1000000
EMBED_DIM = 64
NUM_LABELS = 50
BATCH = 4096
SEQ = 200
PAD_IDX = 3000

NC = 2
NS = 16
NW = NC * NS
BPW = BATCH // NW
SEQP = 256
SEQB = 208
NBLK = SEQB // 16
SPLIT0 = 104
SPLIT1 = SEQ - SPLIT0
NLANE = EMBED_DIM // 16
ROWW = 2 * EMBED_DIM


CHR = 320
NCH = VOCAB // CHR
CPT = -(-NCH // NW)


def _repack_body(table_hbm, out_hbm, vin0, vin1, vout0, vout1,
                 semi0, semi1, semo0, semo1):
  wid = lax.axis_index("s") * NC + lax.axis_index("c")
  semi = {id(vin0): semi0, id(vin1): semi1}
  semo = {id(vout0): semo0, id(vout1): semo1}

  def _in_copy(g, vin):
    return pltpu.make_async_copy(table_hbm.at[pl.ds(g * CHR, CHR)], vin,
                                 semi[id(vin)])

  def _out_copy(g, vout):
    return pltpu.make_async_copy(vout, out_hbm.at[pl.ds(g * (CHR // 2),
                                                        CHR // 2)],
                                 semo[id(vout)])

  def repack(vin, vout):
    def pair_body(p, carry):
      for rr in range(2):
        for c in range(NLANE):
          vout[p, pl.ds(rr * EMBED_DIM + c * 16, 16)] = (
              vin[p * 2 + rr, pl.ds(c * 16, 16)])
      return carry
    lax.fori_loop(0, CHR // 2, pair_body, 0)

  def chunk_of(t):
    return t * NW + wid

  @pl.when(chunk_of(0) < NCH)
  def _():
    _in_copy(chunk_of(0), vin0).start()

  def body2(i, carry):
    t = i * 2
    for tt, vin_a, vout_a, vin_b in ((t, vin0, vout0, vin1),
                                     (t + 1, vin1, vout1, vin0)):
      g = chunk_of(tt)

      @pl.when(chunk_of(tt + 1) < NCH)
      def _():
        _in_copy(chunk_of(tt + 1), vin_b).start()

      @pl.when(g < NCH)
      def _():
        _in_copy(g, vin_a).wait()
        @pl.when(tt >= 2)
        def _():
          _out_copy(chunk_of(tt - 2), vout_a).wait()
        repack(vin_a, vout_a)
        _out_copy(g, vout_a).start()
    return carry

  lax.fori_loop(0, CPT // 2, body2, 0)

  @pl.when(chunk_of(CPT - 2) < NCH)
  def _():
    _out_copy(chunk_of(CPT - 2), vout0).wait()

  @pl.when(chunk_of(CPT - 1) < NCH)
  def _():
    _out_copy(chunk_of(CPT - 1), vout1).wait()


_repack = functools.partial(
    pl.kernel,
    out_type=jax.ShapeDtypeStruct((VOCAB // 2, ROWW), jnp.float32),
    mesh=plsc.VectorSubcoreMesh(core_axis_name="c", subcore_axis_name="s",
                                num_cores=NC, num_subcores=NS),
    scratch_types=[
        pltpu.VMEM((CHR, EMBED_DIM), jnp.float32),
        pltpu.VMEM((CHR, EMBED_DIM), jnp.float32),
        pltpu.VMEM((CHR // 2, ROWW), jnp.float32),
        pltpu.VMEM((CHR // 2, ROWW), jnp.float32),
        pltpu.SemaphoreType.DMA,
        pltpu.SemaphoreType.DMA,
        pltpu.SemaphoreType.DMA,
        pltpu.SemaphoreType.DMA,
    ],
)(_repack_body)


def _sc_pool_body(ids_hbm, table_hbm, out_hbm, idx_v, rows0, rows1,
                  pidx0, pidx1, pooled_v, sem0, sem1):
  wid = lax.axis_index("s") * NC + lax.axis_index("c")
  base = wid * BPW

  pltpu.sync_copy(ids_hbm.at[pl.ds(base * SEQP, BPW * SEQP)], idx_v)

  zeros16 = jnp.zeros((16,), jnp.float32)
  for buf in (rows0, rows1):
    for rz in range(SEQ, SEQB):
      for c in range(ROWW // 16):
        buf[rz, pl.ds(c * 16, 16)] = zeros16

  def _row_copies(r, buf, pidx, sem):
    return (
        pltpu.make_async_copy(table_hbm.at[pidx.at[pl.ds(0, SPLIT0)]],
                              buf.at[pl.ds(0, SPLIT0)], sem),
        pltpu.make_async_copy(table_hbm.at[pidx.at[pl.ds(SPLIT0, SPLIT1)]],
                              buf.at[pl.ds(SPLIT0, SPLIT1)], sem),
    )

  def start_row(r, buf, pidx, sem):
    off = r * SEQP
    for j in range(NBLK):
      iv = idx_v[pl.ds(off + j * 16, 16)]
      pidx[pl.ds(j * 16, 16)] = lax.shift_right_logical(iv, 1)
    for cp in _row_copies(r, buf, pidx, sem):
      cp.start()

  def wait_buf(r, buf, pidx, sem):
    for cp in _row_copies(r, buf, pidx, sem):
      cp.wait()

  def reduce_row(buf, r):
    init = tuple(jnp.full((16,), -jnp.inf, dtype=jnp.float32)
                 for _ in range(NLANE))
    def blk_body(j, accs):
      accs = list(accs)
      l0 = j * 16
      iv = idx_v[pl.ds(r * SEQP + l0, 16)]
      mv = jnp.where(iv == PAD_IDX, jnp.float32(0), jnp.float32(1))
      hv = (iv & 1) * EMBED_DIM
      for u in range(16):
        m = mv[u]
        h = hv[u]
        for c in range(NLANE):
          v = buf[l0 + u, pl.ds(h + c * 16, 16)]
          accs[c] = jnp.maximum(accs[c], v * m)
      return tuple(accs)
    accs = lax.fori_loop(0, NBLK, blk_body, init)
    for c in range(NLANE):
      pooled_v[r, pl.ds(c * 16, 16)] = accs[c]

  start_row(0, rows0, pidx0, sem0)

  def body2(i, carry):
    r = i * 2
    start_row(r + 1, rows1, pidx1, sem1)
    wait_buf(r, rows0, pidx0, sem0)
    reduce_row(rows0, r)

    @pl.when(r + 2 < BPW)
    def _():
      start_row(r + 2, rows0, pidx0, sem0)

    wait_buf(r + 1, rows1, pidx1, sem1)
    reduce_row(rows1, r + 1)
    return carry

  lax.fori_loop(0, BPW // 2, body2, 0)
  pltpu.sync_copy(pooled_v, out_hbm.at[pl.ds(base, BPW)])


_sc_pool = functools.partial(
    pl.kernel,
    out_type=jax.ShapeDtypeStruct((BATCH, EMBED_DIM), jnp.float32),
    mesh=plsc.VectorSubcoreMesh(core_axis_name="c", subcore_axis_name="s",
                                num_cores=NC, num_subcores=NS),
    scratch_types=[
        pltpu.VMEM((BPW * SEQP,), jnp.int32),
        pltpu.VMEM((SEQB, ROWW), jnp.float32),
        pltpu.VMEM((SEQB, ROWW), jnp.float32),
        pltpu.VMEM((SEQB,), jnp.int32),
        pltpu.VMEM((SEQB,), jnp.int32),
        pltpu.VMEM((BPW, EMBED_DIM), jnp.float32),
        pltpu.SemaphoreType.DMA,
        pltpu.SemaphoreType.DMA,
    ],
)(_sc_pool_body)


def _head_body(p_ref, wt_ref, b_ref, o_ref):
  logits = jnp.dot(p_ref[...], wt_ref[...],
                   preferred_element_type=jnp.float32) + b_ref[...]
  mx = jnp.max(logits, axis=1, keepdims=True)
  sh = logits - mx
  lse = jnp.log(jnp.sum(jnp.exp(sh), axis=1, keepdims=True))
  o_ref[...] = sh - lse


_BB = 1024

_head = pl.pallas_call(
    _head_body,
    out_shape=jax.ShapeDtypeStruct((BATCH, NUM_LABELS), jnp.float32),
    grid=(BATCH // _BB,),
    in_specs=[
        pl.BlockSpec((_BB, EMBED_DIM), lambda i: (i, 0)),
        pl.BlockSpec((EMBED_DIM, NUM_LABELS), lambda i: (0, 0)),
        pl.BlockSpec((1, NUM_LABELS), lambda i: (0, 0)),
    ],
    out_specs=pl.BlockSpec((_BB, NUM_LABELS), lambda i: (i, 0)),
)


def kernel(text, sequence_lens, table, W, b):
  del sequence_lens
  ids = jnp.pad(text.astype(jnp.int32), ((0, 0), (0, SEQP - SEQ)),
                constant_values=PAD_IDX).reshape(-1)
  table2 = _repack(table)
  pooled = _sc_pool(ids, table2)
  return _head(pooled, W.T, b.reshape(1, NUM_LABELS))


# speedup vs baseline: 1.1019x; 1.1019x over previous
import functools

import jax
import jax.numpy as jnp
from jax import lax
from jax.experimental import pallas as pl
from jax.experimental.pallas import tpu as pltpu
from jax.experimental.pallas import tpu_sc as plsc

VOCAB = 1000000
EMBED_DIM = 64
NUM_LABELS = 50
BATCH = 4096
SEQ = 200
PAD_IDX = 3000

NC = 2
NS = 16
NW = NC * NS
BPW = BATCH // NW
SEQP = 256
SEQB = 208
NBLK = SEQB // 16
SPLIT0 = 104
SPLIT1 = SEQ - SPLIT0
NLANE = EMBED_DIM // 16
ROWW = 2 * EMBED_DIM


_LB = 512
VOC0 = 500224
NTB = VOC0 // _LB


def _tr_body(t1_ref, t2_ref, o_ref):
  o_ref[:, 0:EMBED_DIM] = t1_ref[...].T
  o_ref[:, EMBED_DIM:2 * EMBED_DIM] = t2_ref[...].T


_transpose = pl.pallas_call(
    _tr_body,
    out_shape=jax.ShapeDtypeStruct((VOC0, 2 * EMBED_DIM), jnp.float32),
    grid=(NTB,),
    in_specs=[
        pl.BlockSpec((EMBED_DIM, _LB), lambda i: (0, i)),
        pl.BlockSpec((EMBED_DIM, _LB), lambda i: (0, NTB + i)),
    ],
    out_specs=pl.BlockSpec((_LB, 2 * EMBED_DIM), lambda i: (i, 0)),
)


CHR = 320
NCH = VOCAB // CHR
CPT = -(-NCH // NW)


def _repack_body(table_hbm, out_hbm, vin0, vin1, vout0, vout1,
                 semi0, semi1, semo0, semo1):
  wid = lax.axis_index("s") * NC + lax.axis_index("c")
  semi = {id(vin0): semi0, id(vin1): semi1}
  semo = {id(vout0): semo0, id(vout1): semo1}

  def _in_copy(g, vin):
    return pltpu.make_async_copy(table_hbm.at[pl.ds(g * CHR, CHR)], vin,
                                 semi[id(vin)])

  def _out_copy(g, vout):
    return pltpu.make_async_copy(vout, out_hbm.at[pl.ds(g * (CHR // 2),
                                                        CHR // 2)],
                                 semo[id(vout)])

  def repack(vin, vout):
    def pair_body(p, carry):
      for rr in range(2):
        for c in range(NLANE):
          vout[p, pl.ds(rr * EMBED_DIM + c * 16, 16)] = (
              vin[p * 2 + rr, pl.ds(c * 16, 16)])
      return carry
    lax.fori_loop(0, CHR // 2, pair_body, 0)

  def chunk_of(t):
    return t * NW + wid

  @pl.when(chunk_of(0) < NCH)
  def _():
    _in_copy(chunk_of(0), vin0).start()

  def body2(i, carry):
    t = i * 2
    for tt, vin_a, vout_a, vin_b in ((t, vin0, vout0, vin1),
                                     (t + 1, vin1, vout1, vin0)):
      g = chunk_of(tt)

      @pl.when(chunk_of(tt + 1) < NCH)
      def _():
        _in_copy(chunk_of(tt + 1), vin_b).start()

      @pl.when(g < NCH)
      def _():
        _in_copy(g, vin_a).wait()
        @pl.when(tt >= 2)
        def _():
          _out_copy(chunk_of(tt - 2), vout_a).wait()
        repack(vin_a, vout_a)
        _out_copy(g, vout_a).start()
    return carry

  lax.fori_loop(0, CPT // 2, body2, 0)

  @pl.when(chunk_of(CPT - 2) < NCH)
  def _():
    _out_copy(chunk_of(CPT - 2), vout0).wait()

  @pl.when(chunk_of(CPT - 1) < NCH)
  def _():
    _out_copy(chunk_of(CPT - 1), vout1).wait()


_repack = functools.partial(
    pl.kernel,
    out_type=jax.ShapeDtypeStruct((VOCAB // 2, ROWW), jnp.float32),
    mesh=plsc.VectorSubcoreMesh(core_axis_name="c", subcore_axis_name="s",
                                num_cores=NC, num_subcores=NS),
    scratch_types=[
        pltpu.VMEM((CHR, EMBED_DIM), jnp.float32),
        pltpu.VMEM((CHR, EMBED_DIM), jnp.float32),
        pltpu.VMEM((CHR // 2, ROWW), jnp.float32),
        pltpu.VMEM((CHR // 2, ROWW), jnp.float32),
        pltpu.SemaphoreType.DMA,
        pltpu.SemaphoreType.DMA,
        pltpu.SemaphoreType.DMA,
        pltpu.SemaphoreType.DMA,
    ],
)(_repack_body)


def _sc_pool_body(ids_hbm, table_hbm, out_hbm, idx_v, rows0, rows1,
                  pidx0, pidx1, pooled_v, sem0, sem1):
  wid = lax.axis_index("s") * NC + lax.axis_index("c")
  base = wid * BPW

  pltpu.sync_copy(ids_hbm.at[pl.ds(base * SEQP, BPW * SEQP)], idx_v)

  zeros16 = jnp.zeros((16,), jnp.float32)
  for buf in (rows0, rows1):
    for rz in range(SEQ, SEQB):
      for c in range(ROWW // 16):
        buf[rz, pl.ds(c * 16, 16)] = zeros16

  def _row_copies(r, buf, pidx, sem):
    return (
        pltpu.make_async_copy(table_hbm.at[pidx.at[pl.ds(0, SPLIT0)]],
                              buf.at[pl.ds(0, SPLIT0)], sem),
        pltpu.make_async_copy(table_hbm.at[pidx.at[pl.ds(SPLIT0, SPLIT1)]],
                              buf.at[pl.ds(SPLIT0, SPLIT1)], sem),
    )

  def start_row(r, buf, pidx, sem):
    off = r * SEQP
    for j in range(NBLK):
      iv = idx_v[pl.ds(off + j * 16, 16)]
      pidx[pl.ds(j * 16, 16)] = iv - jnp.where(iv >= VOC0, VOC0, 0)
    for cp in _row_copies(r, buf, pidx, sem):
      cp.start()

  def wait_buf(r, buf, pidx, sem):
    for cp in _row_copies(r, buf, pidx, sem):
      cp.wait()

  def reduce_row(buf, r):
    init = tuple(jnp.full((16,), -jnp.inf, dtype=jnp.float32)
                 for _ in range(NLANE))
    def blk_body(j, accs):
      accs = list(accs)
      l0 = j * 16
      iv = idx_v[pl.ds(r * SEQP + l0, 16)]
      mv = jnp.where(iv == PAD_IDX, jnp.float32(0), jnp.float32(1))
      hv = jnp.where(iv >= VOC0, EMBED_DIM, 0)
      for u in range(16):
        m = mv[u]
        h = hv[u]
        for c in range(NLANE):
          v = buf[l0 + u, pl.ds(h + c * 16, 16)]
          accs[c] = jnp.maximum(accs[c], v * m)
      return tuple(accs)
    accs = lax.fori_loop(0, NBLK, blk_body, init)
    for c in range(NLANE):
      pooled_v[r, pl.ds(c * 16, 16)] = accs[c]

  start_row(0, rows0, pidx0, sem0)

  def body2(i, carry):
    r = i * 2
    start_row(r + 1, rows1, pidx1, sem1)
    wait_buf(r, rows0, pidx0, sem0)
    reduce_row(rows0, r)

    @pl.when(r + 2 < BPW)
    def _():
      start_row(r + 2, rows0, pidx0, sem0)

    wait_buf(r + 1, rows1, pidx1, sem1)
    reduce_row(rows1, r + 1)
    return carry

  lax.fori_loop(0, BPW // 2, body2, 0)
  pltpu.sync_copy(pooled_v, out_hbm.at[pl.ds(base, BPW)])


_sc_pool = functools.partial(
    pl.kernel,
    out_type=jax.ShapeDtypeStruct((BATCH, EMBED_DIM), jnp.float32),
    mesh=plsc.VectorSubcoreMesh(core_axis_name="c", subcore_axis_name="s",
                                num_cores=NC, num_subcores=NS),
    scratch_types=[
        pltpu.VMEM((BPW * SEQP,), jnp.int32),
        pltpu.VMEM((SEQB, ROWW), jnp.float32),
        pltpu.VMEM((SEQB, ROWW), jnp.float32),
        pltpu.VMEM((SEQB,), jnp.int32),
        pltpu.VMEM((SEQB,), jnp.int32),
        pltpu.VMEM((BPW, EMBED_DIM), jnp.float32),
        pltpu.SemaphoreType.DMA,
        pltpu.SemaphoreType.DMA,
    ],
)(_sc_pool_body)


def _head_body(p_ref, wt_ref, b_ref, o_ref):
  logits = jnp.dot(p_ref[...], wt_ref[...],
                   preferred_element_type=jnp.float32) + b_ref[...]
  mx = jnp.max(logits, axis=1, keepdims=True)
  sh = logits - mx
  lse = jnp.log(jnp.sum(jnp.exp(sh), axis=1, keepdims=True))
  o_ref[...] = sh - lse


_BB = 1024

_head = pl.pallas_call(
    _head_body,
    out_shape=jax.ShapeDtypeStruct((BATCH, NUM_LABELS), jnp.float32),
    grid=(BATCH // _BB,),
    in_specs=[
        pl.BlockSpec((_BB, EMBED_DIM), lambda i: (i, 0)),
        pl.BlockSpec((EMBED_DIM, NUM_LABELS), lambda i: (0, 0)),
        pl.BlockSpec((1, NUM_LABELS), lambda i: (0, 0)),
    ],
    out_specs=pl.BlockSpec((_BB, NUM_LABELS), lambda i: (i, 0)),
)


def kernel(text, sequence_lens, table, W, b):
  del sequence_lens
  ids = jnp.pad(text.astype(jnp.int32), ((0, 0), (0, SEQP - SEQ)),
                constant_values=PAD_IDX).reshape(-1)
  tt = table.T
  table2 = _transpose(tt, tt)
  pooled = _sc_pool(ids, table2)
  return _head(pooled, W.T, b.reshape(1, NUM_LABELS))


# speedup vs baseline: 1.8396x; 1.6695x over previous
import functools

import jax
import jax.numpy as jnp
from jax import lax
from jax.experimental import pallas as pl
from jax.experimental.pallas import tpu as pltpu
from jax.experimental.pallas import tpu_sc as plsc

VOCAB = 1000000
EMBED_DIM = 64
NUM_LABELS = 50
BATCH = 4096
SEQ = 200
PAD_IDX = 3000

NC = 2
NS = 16
NW = NC * NS
BPW = BATCH // NW
SEQP = 256
SEQB = 208
NBLK = SEQB // 16
SPLIT0 = 104
SPLIT1 = SEQ - SPLIT0
NLANE = EMBED_DIM // 16
ROWW = 2 * EMBED_DIM


_LB = 2048
NTB = 245
VOC0 = NTB * _LB
_NIN = -(-VOCAB // _LB)


def _tr_body(t1_ref, t2_ref, o_ref):
  o_ref[:, 0:EMBED_DIM] = t1_ref[...].T
  o_ref[:, EMBED_DIM:2 * EMBED_DIM] = t2_ref[...].T


_transpose = pl.pallas_call(
    _tr_body,
    out_shape=jax.ShapeDtypeStruct((VOC0, 2 * EMBED_DIM), jnp.float32),
    grid=(NTB,),
    in_specs=[
        pl.BlockSpec((EMBED_DIM, _LB), lambda i: (0, i)),
        pl.BlockSpec((EMBED_DIM, _LB),
                     lambda i: (0, jnp.minimum(NTB + i, _NIN - 1))),
    ],
    out_specs=pl.BlockSpec((_LB, 2 * EMBED_DIM), lambda i: (i, 0)),
)


CHR = 320
NCH = VOCAB // CHR
CPT = -(-NCH // NW)


def _repack_body(table_hbm, out_hbm, vin0, vin1, vout0, vout1,
                 semi0, semi1, semo0, semo1):
  wid = lax.axis_index("s") * NC + lax.axis_index("c")
  semi = {id(vin0): semi0, id(vin1): semi1}
  semo = {id(vout0): semo0, id(vout1): semo1}

  def _in_copy(g, vin):
    return pltpu.make_async_copy(table_hbm.at[pl.ds(g * CHR, CHR)], vin,
                                 semi[id(vin)])

  def _out_copy(g, vout):
    return pltpu.make_async_copy(vout, out_hbm.at[pl.ds(g * (CHR // 2),
                                                        CHR // 2)],
                                 semo[id(vout)])

  def repack(vin, vout):
    def pair_body(p, carry):
      for rr in range(2):
        for c in range(NLANE):
          vout[p, pl.ds(rr * EMBED_DIM + c * 16, 16)] = (
              vin[p * 2 + rr, pl.ds(c * 16, 16)])
      return carry
    lax.fori_loop(0, CHR // 2, pair_body, 0)

  def chunk_of(t):
    return t * NW + wid

  @pl.when(chunk_of(0) < NCH)
  def _():
    _in_copy(chunk_of(0), vin0).start()

  def body2(i, carry):
    t = i * 2
    for tt, vin_a, vout_a, vin_b in ((t, vin0, vout0, vin1),
                                     (t + 1, vin1, vout1, vin0)):
      g = chunk_of(tt)

      @pl.when(chunk_of(tt + 1) < NCH)
      def _():
        _in_copy(chunk_of(tt + 1), vin_b).start()

      @pl.when(g < NCH)
      def _():
        _in_copy(g, vin_a).wait()
        @pl.when(tt >= 2)
        def _():
          _out_copy(chunk_of(tt - 2), vout_a).wait()
        repack(vin_a, vout_a)
        _out_copy(g, vout_a).start()
    return carry

  lax.fori_loop(0, CPT // 2, body2, 0)

  @pl.when(chunk_of(CPT - 2) < NCH)
  def _():
    _out_copy(chunk_of(CPT - 2), vout0).wait()

  @pl.when(chunk_of(CPT - 1) < NCH)
  def _():
    _out_copy(chunk_of(CPT - 1), vout1).wait()


_repack = functools.partial(
    pl.kernel,
    out_type=jax.ShapeDtypeStruct((VOCAB // 2, ROWW), jnp.float32),
    mesh=plsc.VectorSubcoreMesh(core_axis_name="c", subcore_axis_name="s",
                                num_cores=NC, num_subcores=NS),
    scratch_types=[
        pltpu.VMEM((CHR, EMBED_DIM), jnp.float32),
        pltpu.VMEM((CHR, EMBED_DIM), jnp.float32),
        pltpu.VMEM((CHR // 2, ROWW), jnp.float32),
        pltpu.VMEM((CHR // 2, ROWW), jnp.float32),
        pltpu.SemaphoreType.DMA,
        pltpu.SemaphoreType.DMA,
        pltpu.SemaphoreType.DMA,
        pltpu.SemaphoreType.DMA,
    ],
)(_repack_body)


def _sc_pool_body(ids_hbm, table_hbm, out_hbm, idx_v, rows0, rows1,
                  pidx0, pidx1, pooled_v, sem0, sem1):
  wid = lax.axis_index("s") * NC + lax.axis_index("c")
  base = wid * BPW

  pltpu.sync_copy(ids_hbm.at[pl.ds(base * SEQP, BPW * SEQP)], idx_v)

  zeros16 = jnp.zeros((16,), jnp.float32)
  for buf in (rows0, rows1):
    for rz in range(SEQ, SEQB):
      for c in range(ROWW // 16):
        buf[rz, pl.ds(c * 16, 16)] = zeros16

  def _row_copies(r, buf, pidx, sem):
    return (
        pltpu.make_async_copy(table_hbm.at[pidx.at[pl.ds(0, SPLIT0)]],
                              buf.at[pl.ds(0, SPLIT0)], sem),
        pltpu.make_async_copy(table_hbm.at[pidx.at[pl.ds(SPLIT0, SPLIT1)]],
                              buf.at[pl.ds(SPLIT0, SPLIT1)], sem),
    )

  def start_row(r, buf, pidx, sem):
    off = r * SEQP
    for j in range(NBLK):
      iv = idx_v[pl.ds(off + j * 16, 16)]
      pidx[pl.ds(j * 16, 16)] = iv - jnp.where(iv >= VOC0, VOC0, 0)
    for cp in _row_copies(r, buf, pidx, sem):
      cp.start()

  def wait_buf(r, buf, pidx, sem):
    for cp in _row_copies(r, buf, pidx, sem):
      cp.wait()

  def reduce_row(buf, r):
    init = tuple(jnp.full((16,), -jnp.inf, dtype=jnp.float32)
                 for _ in range(NLANE))
    def blk_body(j, accs):
      accs = list(accs)
      l0 = j * 16
      iv = idx_v[pl.ds(r * SEQP + l0, 16)]
      mv = jnp.where(iv == PAD_IDX, jnp.float32(0), jnp.float32(1))
      hv = jnp.where(iv >= VOC0, EMBED_DIM, 0)
      for u in range(16):
        m = mv[u]
        h = hv[u]
        for c in range(NLANE):
          v = buf[l0 + u, pl.ds(h + c * 16, 16)]
          accs[c] = jnp.maximum(accs[c], v * m)
      return tuple(accs)
    accs = lax.fori_loop(0, NBLK, blk_body, init)
    for c in range(NLANE):
      pooled_v[r, pl.ds(c * 16, 16)] = accs[c]

  start_row(0, rows0, pidx0, sem0)

  def body2(i, carry):
    r = i * 2
    start_row(r + 1, rows1, pidx1, sem1)
    wait_buf(r, rows0, pidx0, sem0)
    reduce_row(rows0, r)

    @pl.when(r + 2 < BPW)
    def _():
      start_row(r + 2, rows0, pidx0, sem0)

    wait_buf(r + 1, rows1, pidx1, sem1)
    reduce_row(rows1, r + 1)
    return carry

  lax.fori_loop(0, BPW // 2, body2, 0)
  pltpu.sync_copy(pooled_v, out_hbm.at[pl.ds(base, BPW)])


_sc_pool = functools.partial(
    pl.kernel,
    out_type=jax.ShapeDtypeStruct((BATCH, EMBED_DIM), jnp.float32),
    mesh=plsc.VectorSubcoreMesh(core_axis_name="c", subcore_axis_name="s",
                                num_cores=NC, num_subcores=NS),
    scratch_types=[
        pltpu.VMEM((BPW * SEQP,), jnp.int32),
        pltpu.VMEM((SEQB, ROWW), jnp.float32),
        pltpu.VMEM((SEQB, ROWW), jnp.float32),
        pltpu.VMEM((SEQB,), jnp.int32),
        pltpu.VMEM((SEQB,), jnp.int32),
        pltpu.VMEM((BPW, EMBED_DIM), jnp.float32),
        pltpu.SemaphoreType.DMA,
        pltpu.SemaphoreType.DMA,
    ],
)(_sc_pool_body)


def _head_body(p_ref, wt_ref, b_ref, o_ref):
  logits = jnp.dot(p_ref[...], wt_ref[...],
                   preferred_element_type=jnp.float32) + b_ref[...]
  mx = jnp.max(logits, axis=1, keepdims=True)
  sh = logits - mx
  lse = jnp.log(jnp.sum(jnp.exp(sh), axis=1, keepdims=True))
  o_ref[...] = sh - lse


_BB = 1024

_head = pl.pallas_call(
    _head_body,
    out_shape=jax.ShapeDtypeStruct((BATCH, NUM_LABELS), jnp.float32),
    grid=(BATCH // _BB,),
    in_specs=[
        pl.BlockSpec((_BB, EMBED_DIM), lambda i: (i, 0)),
        pl.BlockSpec((EMBED_DIM, NUM_LABELS), lambda i: (0, 0)),
        pl.BlockSpec((1, NUM_LABELS), lambda i: (0, 0)),
    ],
    out_specs=pl.BlockSpec((_BB, NUM_LABELS), lambda i: (i, 0)),
)


def kernel(text, sequence_lens, table, W, b):
  del sequence_lens
  ids = jnp.pad(text.astype(jnp.int32), ((0, 0), (0, SEQP - SEQ)),
                constant_values=PAD_IDX).reshape(-1)
  tt = table.T
  table2 = _transpose(tt, tt)
  pooled = _sc_pool(ids, table2)
  return _head(pooled, W.T, b.reshape(1, NUM_LABELS))


# speedup vs baseline: 2.0893x; 1.1357x over previous
import functools

import jax
import jax.numpy as jnp
from jax import lax
from jax.experimental import pallas as pl
from jax.experimental.pallas import tpu as pltpu
from jax.experimental.pallas import tpu_sc as plsc

VOCAB = 1000000
EMBED_DIM = 64
NUM_LABELS = 50
BATCH = 4096
SEQ = 200
PAD_IDX = 3000

NC = 2
NS = 16
NW = NC * NS
BPW = BATCH // NW
SEQP = 256
SEQB = 208
NBLK = SEQB // 16
SPLIT0 = 104
SPLIT1 = SEQ - SPLIT0
NLANE = EMBED_DIM // 16
ROWW = 2 * EMBED_DIM


_LB = 4096
NTB = 123
VOC0 = NTB * _LB
_NIN = -(-VOCAB // _LB)


def _tr_body(t1_ref, t2_ref, o_ref):
  o_ref[:, 0:EMBED_DIM] = t1_ref[...].T
  o_ref[:, EMBED_DIM:2 * EMBED_DIM] = t2_ref[...].T


_transpose = pl.pallas_call(
    _tr_body,
    out_shape=jax.ShapeDtypeStruct((VOC0, 2 * EMBED_DIM), jnp.float32),
    grid=(NTB,),
    in_specs=[
        pl.BlockSpec((EMBED_DIM, _LB), lambda i: (0, i)),
        pl.BlockSpec((EMBED_DIM, _LB),
                     lambda i: (0, jnp.minimum(NTB + i, _NIN - 1))),
    ],
    out_specs=pl.BlockSpec((_LB, 2 * EMBED_DIM), lambda i: (i, 0)),
)


CHR = 320
NCH = VOCAB // CHR
CPT = -(-NCH // NW)


def _repack_body(table_hbm, out_hbm, vin0, vin1, vout0, vout1,
                 semi0, semi1, semo0, semo1):
  wid = lax.axis_index("s") * NC + lax.axis_index("c")
  semi = {id(vin0): semi0, id(vin1): semi1}
  semo = {id(vout0): semo0, id(vout1): semo1}

  def _in_copy(g, vin):
    return pltpu.make_async_copy(table_hbm.at[pl.ds(g * CHR, CHR)], vin,
                                 semi[id(vin)])

  def _out_copy(g, vout):
    return pltpu.make_async_copy(vout, out_hbm.at[pl.ds(g * (CHR // 2),
                                                        CHR // 2)],
                                 semo[id(vout)])

  def repack(vin, vout):
    def pair_body(p, carry):
      for rr in range(2):
        for c in range(NLANE):
          vout[p, pl.ds(rr * EMBED_DIM + c * 16, 16)] = (
              vin[p * 2 + rr, pl.ds(c * 16, 16)])
      return carry
    lax.fori_loop(0, CHR // 2, pair_body, 0)

  def chunk_of(t):
    return t * NW + wid

  @pl.when(chunk_of(0) < NCH)
  def _():
    _in_copy(chunk_of(0), vin0).start()

  def body2(i, carry):
    t = i * 2
    for tt, vin_a, vout_a, vin_b in ((t, vin0, vout0, vin1),
                                     (t + 1, vin1, vout1, vin0)):
      g = chunk_of(tt)

      @pl.when(chunk_of(tt + 1) < NCH)
      def _():
        _in_copy(chunk_of(tt + 1), vin_b).start()

      @pl.when(g < NCH)
      def _():
        _in_copy(g, vin_a).wait()
        @pl.when(tt >= 2)
        def _():
          _out_copy(chunk_of(tt - 2), vout_a).wait()
        repack(vin_a, vout_a)
        _out_copy(g, vout_a).start()
    return carry

  lax.fori_loop(0, CPT // 2, body2, 0)

  @pl.when(chunk_of(CPT - 2) < NCH)
  def _():
    _out_copy(chunk_of(CPT - 2), vout0).wait()

  @pl.when(chunk_of(CPT - 1) < NCH)
  def _():
    _out_copy(chunk_of(CPT - 1), vout1).wait()


_repack = functools.partial(
    pl.kernel,
    out_type=jax.ShapeDtypeStruct((VOCAB // 2, ROWW), jnp.float32),
    mesh=plsc.VectorSubcoreMesh(core_axis_name="c", subcore_axis_name="s",
                                num_cores=NC, num_subcores=NS),
    scratch_types=[
        pltpu.VMEM((CHR, EMBED_DIM), jnp.float32),
        pltpu.VMEM((CHR, EMBED_DIM), jnp.float32),
        pltpu.VMEM((CHR // 2, ROWW), jnp.float32),
        pltpu.VMEM((CHR // 2, ROWW), jnp.float32),
        pltpu.SemaphoreType.DMA,
        pltpu.SemaphoreType.DMA,
        pltpu.SemaphoreType.DMA,
        pltpu.SemaphoreType.DMA,
    ],
)(_repack_body)


def _sc_pool_body(ids_hbm, table_hbm, out_hbm, idx_v, rows0, rows1,
                  pidx0, pidx1, pooled_v, sem0, sem1):
  wid = lax.axis_index("s") * NC + lax.axis_index("c")
  base = wid * BPW

  pltpu.sync_copy(ids_hbm.at[pl.ds(base * SEQP, BPW * SEQP)], idx_v)

  zeros16 = jnp.zeros((16,), jnp.float32)
  for buf in (rows0, rows1):
    for rz in range(SEQ, SEQB):
      for c in range(ROWW // 16):
        buf[rz, pl.ds(c * 16, 16)] = zeros16

  def _row_copies(r, buf, pidx, sem):
    return (
        pltpu.make_async_copy(table_hbm.at[pidx.at[pl.ds(0, SPLIT0)]],
                              buf.at[pl.ds(0, SPLIT0)], sem),
        pltpu.make_async_copy(table_hbm.at[pidx.at[pl.ds(SPLIT0, SPLIT1)]],
                              buf.at[pl.ds(SPLIT0, SPLIT1)], sem),
    )

  def start_row(r, buf, pidx, sem):
    off = r * SEQP
    for j in range(NBLK):
      iv = idx_v[pl.ds(off + j * 16, 16)]
      pidx[pl.ds(j * 16, 16)] = iv - jnp.where(iv >= VOC0, VOC0, 0)
    for cp in _row_copies(r, buf, pidx, sem):
      cp.start()

  def wait_buf(r, buf, pidx, sem):
    for cp in _row_copies(r, buf, pidx, sem):
      cp.wait()

  def reduce_row(buf, r):
    init = tuple(jnp.full((16,), -jnp.inf, dtype=jnp.float32)
                 for _ in range(NLANE))
    def blk_body(j, accs):
      accs = list(accs)
      l0 = j * 16
      iv = idx_v[pl.ds(r * SEQP + l0, 16)]
      mv = jnp.where(iv == PAD_IDX, jnp.float32(0), jnp.float32(1))
      hv = jnp.where(iv >= VOC0, EMBED_DIM, 0)
      for u in range(16):
        m = mv[u]
        h = hv[u]
        for c in range(NLANE):
          v = buf[l0 + u, pl.ds(h + c * 16, 16)]
          accs[c] = jnp.maximum(accs[c], v * m)
      return tuple(accs)
    accs = lax.fori_loop(0, NBLK, blk_body, init)
    for c in range(NLANE):
      pooled_v[r, pl.ds(c * 16, 16)] = accs[c]

  start_row(0, rows0, pidx0, sem0)

  def body2(i, carry):
    r = i * 2
    start_row(r + 1, rows1, pidx1, sem1)
    wait_buf(r, rows0, pidx0, sem0)
    reduce_row(rows0, r)

    @pl.when(r + 2 < BPW)
    def _():
      start_row(r + 2, rows0, pidx0, sem0)

    wait_buf(r + 1, rows1, pidx1, sem1)
    reduce_row(rows1, r + 1)
    return carry

  lax.fori_loop(0, BPW // 2, body2, 0)
  pltpu.sync_copy(pooled_v, out_hbm.at[pl.ds(base, BPW)])


_sc_pool = functools.partial(
    pl.kernel,
    out_type=jax.ShapeDtypeStruct((BATCH, EMBED_DIM), jnp.float32),
    mesh=plsc.VectorSubcoreMesh(core_axis_name="c", subcore_axis_name="s",
                                num_cores=NC, num_subcores=NS),
    scratch_types=[
        pltpu.VMEM((BPW * SEQP,), jnp.int32),
        pltpu.VMEM((SEQB, ROWW), jnp.float32),
        pltpu.VMEM((SEQB, ROWW), jnp.float32),
        pltpu.VMEM((SEQB,), jnp.int32),
        pltpu.VMEM((SEQB,), jnp.int32),
        pltpu.VMEM((BPW, EMBED_DIM), jnp.float32),
        pltpu.SemaphoreType.DMA,
        pltpu.SemaphoreType.DMA,
    ],
)(_sc_pool_body)


def _head_body(p_ref, wt_ref, b_ref, o_ref):
  logits = jnp.dot(p_ref[...], wt_ref[...],
                   preferred_element_type=jnp.float32) + b_ref[...]
  mx = jnp.max(logits, axis=1, keepdims=True)
  sh = logits - mx
  lse = jnp.log(jnp.sum(jnp.exp(sh), axis=1, keepdims=True))
  o_ref[...] = sh - lse


_BB = 1024

_head = pl.pallas_call(
    _head_body,
    out_shape=jax.ShapeDtypeStruct((BATCH, NUM_LABELS), jnp.float32),
    grid=(BATCH // _BB,),
    in_specs=[
        pl.BlockSpec((_BB, EMBED_DIM), lambda i: (i, 0)),
        pl.BlockSpec((EMBED_DIM, NUM_LABELS), lambda i: (0, 0)),
        pl.BlockSpec((1, NUM_LABELS), lambda i: (0, 0)),
    ],
    out_specs=pl.BlockSpec((_BB, NUM_LABELS), lambda i: (i, 0)),
)


def kernel(text, sequence_lens, table, W, b):
  del sequence_lens
  ids = jnp.pad(text.astype(jnp.int32), ((0, 0), (0, SEQP - SEQ)),
                constant_values=PAD_IDX).reshape(-1)
  tt = table.T
  table2 = _transpose(tt, tt)
  pooled = _sc_pool(ids, table2)
  return _head(pooled, W.T, b.reshape(1, NUM_LABELS))


# speedup vs baseline: 2.2471x; 1.0755x over previous
import functools

import jax
import jax.numpy as jnp
from jax import lax
from jax.experimental import pallas as pl
from jax.experimental.pallas import tpu as pltpu
from jax.experimental.pallas import tpu_sc as plsc

VOCAB = 1000000
EMBED_DIM = 64
NUM_LABELS = 50
BATCH = 4096
SEQ = 200
PAD_IDX = 3000

NC = 2
NS = 16
NW = NC * NS
BPW = BATCH // NW
SEQP = 256
SEQB = 208
NBLK = SEQB // 16
SPLIT0 = 104
SPLIT1 = SEQ - SPLIT0
NLANE = EMBED_DIM // 16
ROWW = 2 * EMBED_DIM


_LB = 8192
NTB = 62
VOC0 = NTB * _LB
_NIN = -(-VOCAB // _LB)


def _tr_body(t1_ref, t2_ref, o_ref):
  o_ref[:, 0:EMBED_DIM] = t1_ref[...].T
  o_ref[:, EMBED_DIM:2 * EMBED_DIM] = t2_ref[...].T


_transpose = pl.pallas_call(
    _tr_body,
    out_shape=jax.ShapeDtypeStruct((VOC0, 2 * EMBED_DIM), jnp.float32),
    grid=(NTB,),
    in_specs=[
        pl.BlockSpec((EMBED_DIM, _LB), lambda i: (0, i)),
        pl.BlockSpec((EMBED_DIM, _LB),
                     lambda i: (0, jnp.minimum(NTB + i, _NIN - 1))),
    ],
    out_specs=pl.BlockSpec((_LB, 2 * EMBED_DIM), lambda i: (i, 0)),
)


CHR = 320
NCH = VOCAB // CHR
CPT = -(-NCH // NW)


def _repack_body(table_hbm, out_hbm, vin0, vin1, vout0, vout1,
                 semi0, semi1, semo0, semo1):
  wid = lax.axis_index("s") * NC + lax.axis_index("c")
  semi = {id(vin0): semi0, id(vin1): semi1}
  semo = {id(vout0): semo0, id(vout1): semo1}

  def _in_copy(g, vin):
    return pltpu.make_async_copy(table_hbm.at[pl.ds(g * CHR, CHR)], vin,
                                 semi[id(vin)])

  def _out_copy(g, vout):
    return pltpu.make_async_copy(vout, out_hbm.at[pl.ds(g * (CHR // 2),
                                                        CHR // 2)],
                                 semo[id(vout)])

  def repack(vin, vout):
    def pair_body(p, carry):
      for rr in range(2):
        for c in range(NLANE):
          vout[p, pl.ds(rr * EMBED_DIM + c * 16, 16)] = (
              vin[p * 2 + rr, pl.ds(c * 16, 16)])
      return carry
    lax.fori_loop(0, CHR // 2, pair_body, 0)

  def chunk_of(t):
    return t * NW + wid

  @pl.when(chunk_of(0) < NCH)
  def _():
    _in_copy(chunk_of(0), vin0).start()

  def body2(i, carry):
    t = i * 2
    for tt, vin_a, vout_a, vin_b in ((t, vin0, vout0, vin1),
                                     (t + 1, vin1, vout1, vin0)):
      g = chunk_of(tt)

      @pl.when(chunk_of(tt + 1) < NCH)
      def _():
        _in_copy(chunk_of(tt + 1), vin_b).start()

      @pl.when(g < NCH)
      def _():
        _in_copy(g, vin_a).wait()
        @pl.when(tt >= 2)
        def _():
          _out_copy(chunk_of(tt - 2), vout_a).wait()
        repack(vin_a, vout_a)
        _out_copy(g, vout_a).start()
    return carry

  lax.fori_loop(0, CPT // 2, body2, 0)

  @pl.when(chunk_of(CPT - 2) < NCH)
  def _():
    _out_copy(chunk_of(CPT - 2), vout0).wait()

  @pl.when(chunk_of(CPT - 1) < NCH)
  def _():
    _out_copy(chunk_of(CPT - 1), vout1).wait()


_repack = functools.partial(
    pl.kernel,
    out_type=jax.ShapeDtypeStruct((VOCAB // 2, ROWW), jnp.float32),
    mesh=plsc.VectorSubcoreMesh(core_axis_name="c", subcore_axis_name="s",
                                num_cores=NC, num_subcores=NS),
    scratch_types=[
        pltpu.VMEM((CHR, EMBED_DIM), jnp.float32),
        pltpu.VMEM((CHR, EMBED_DIM), jnp.float32),
        pltpu.VMEM((CHR // 2, ROWW), jnp.float32),
        pltpu.VMEM((CHR // 2, ROWW), jnp.float32),
        pltpu.SemaphoreType.DMA,
        pltpu.SemaphoreType.DMA,
        pltpu.SemaphoreType.DMA,
        pltpu.SemaphoreType.DMA,
    ],
)(_repack_body)


def _sc_pool_body(ids_hbm, table_hbm, out_hbm, idx_v, rows0, rows1,
                  pidx0, pidx1, pooled_v, sem0, sem1):
  wid = lax.axis_index("s") * NC + lax.axis_index("c")
  base = wid * BPW

  pltpu.sync_copy(ids_hbm.at[pl.ds(base * SEQP, BPW * SEQP)], idx_v)

  zeros16 = jnp.zeros((16,), jnp.float32)
  for buf in (rows0, rows1):
    for rz in range(SEQ, SEQB):
      for c in range(ROWW // 16):
        buf[rz, pl.ds(c * 16, 16)] = zeros16

  def _row_copies(r, buf, pidx, sem):
    return (
        pltpu.make_async_copy(table_hbm.at[pidx.at[pl.ds(0, SPLIT0)]],
                              buf.at[pl.ds(0, SPLIT0)], sem),
        pltpu.make_async_copy(table_hbm.at[pidx.at[pl.ds(SPLIT0, SPLIT1)]],
                              buf.at[pl.ds(SPLIT0, SPLIT1)], sem),
    )

  def start_row(r, buf, pidx, sem):
    off = r * SEQP
    for j in range(NBLK):
      iv = idx_v[pl.ds(off + j * 16, 16)]
      pidx[pl.ds(j * 16, 16)] = iv - jnp.where(iv >= VOC0, VOC0, 0)
    for cp in _row_copies(r, buf, pidx, sem):
      cp.start()

  def wait_buf(r, buf, pidx, sem):
    for cp in _row_copies(r, buf, pidx, sem):
      cp.wait()

  def reduce_row(buf, r):
    init = tuple(jnp.full((16,), -jnp.inf, dtype=jnp.float32)
                 for _ in range(NLANE))
    def blk_body(j, accs):
      accs = list(accs)
      l0 = j * 16
      iv = idx_v[pl.ds(r * SEQP + l0, 16)]
      mv = jnp.where(iv == PAD_IDX, jnp.float32(0), jnp.float32(1))
      hv = jnp.where(iv >= VOC0, EMBED_DIM, 0)
      for u in range(16):
        m = mv[u]
        h = hv[u]
        for c in range(NLANE):
          v = buf[l0 + u, pl.ds(h + c * 16, 16)]
          accs[c] = jnp.maximum(accs[c], v * m)
      return tuple(accs)
    accs = lax.fori_loop(0, NBLK, blk_body, init)
    for c in range(NLANE):
      pooled_v[r, pl.ds(c * 16, 16)] = accs[c]

  start_row(0, rows0, pidx0, sem0)

  def body2(i, carry):
    r = i * 2
    start_row(r + 1, rows1, pidx1, sem1)
    wait_buf(r, rows0, pidx0, sem0)
    reduce_row(rows0, r)

    @pl.when(r + 2 < BPW)
    def _():
      start_row(r + 2, rows0, pidx0, sem0)

    wait_buf(r + 1, rows1, pidx1, sem1)
    reduce_row(rows1, r + 1)
    return carry

  lax.fori_loop(0, BPW // 2, body2, 0)
  pltpu.sync_copy(pooled_v, out_hbm.at[pl.ds(base, BPW)])


_sc_pool = functools.partial(
    pl.kernel,
    out_type=jax.ShapeDtypeStruct((BATCH, EMBED_DIM), jnp.float32),
    mesh=plsc.VectorSubcoreMesh(core_axis_name="c", subcore_axis_name="s",
                                num_cores=NC, num_subcores=NS),
    scratch_types=[
        pltpu.VMEM((BPW * SEQP,), jnp.int32),
        pltpu.VMEM((SEQB, ROWW), jnp.float32),
        pltpu.VMEM((SEQB, ROWW), jnp.float32),
        pltpu.VMEM((SEQB,), jnp.int32),
        pltpu.VMEM((SEQB,), jnp.int32),
        pltpu.VMEM((BPW, EMBED_DIM), jnp.float32),
        pltpu.SemaphoreType.DMA,
        pltpu.SemaphoreType.DMA,
    ],
)(_sc_pool_body)


def _head_body(p_ref, wt_ref, b_ref, o_ref):
  logits = jnp.dot(p_ref[...], wt_ref[...],
                   preferred_element_type=jnp.float32) + b_ref[...]
  mx = jnp.max(logits, axis=1, keepdims=True)
  sh = logits - mx
  lse = jnp.log(jnp.sum(jnp.exp(sh), axis=1, keepdims=True))
  o_ref[...] = sh - lse


_BB = 1024

_head = pl.pallas_call(
    _head_body,
    out_shape=jax.ShapeDtypeStruct((BATCH, NUM_LABELS), jnp.float32),
    grid=(BATCH // _BB,),
    in_specs=[
        pl.BlockSpec((_BB, EMBED_DIM), lambda i: (i, 0)),
        pl.BlockSpec((EMBED_DIM, NUM_LABELS), lambda i: (0, 0)),
        pl.BlockSpec((1, NUM_LABELS), lambda i: (0, 0)),
    ],
    out_specs=pl.BlockSpec((_BB, NUM_LABELS), lambda i: (i, 0)),
)


def kernel(text, sequence_lens, table, W, b):
  del sequence_lens
  ids = jnp.pad(text.astype(jnp.int32), ((0, 0), (0, SEQP - SEQ)),
                constant_values=PAD_IDX).reshape(-1)
  tt = table.T
  table2 = _transpose(tt, tt)
  pooled = _sc_pool(ids, table2)
  return _head(pooled, W.T, b.reshape(1, NUM_LABELS))


# speedup vs baseline: 2.3062x; 1.0263x over previous
import functools

import jax
import jax.numpy as jnp
from jax import lax
from jax.experimental import pallas as pl
from jax.experimental.pallas import tpu as pltpu
from jax.experimental.pallas import tpu_sc as plsc

VOCAB = 1000000
EMBED_DIM = 64
NUM_LABELS = 50
BATCH = 4096
SEQ = 200
PAD_IDX = 3000

NC = 2
NS = 16
NW = NC * NS
BPW = BATCH // NW
SEQP = 256
SEQB = 208
NBLK = SEQB // 16
SPLIT0 = 104
SPLIT1 = SEQ - SPLIT0
NLANE = EMBED_DIM // 16
ROWW = 2 * EMBED_DIM


_LB = 16384
NTB = 31
VOC0 = NTB * _LB
_NIN = -(-VOCAB // _LB)


def _tr_body(t1_ref, t2_ref, o_ref):
  o_ref[:, 0:EMBED_DIM] = t1_ref[...].T
  o_ref[:, EMBED_DIM:2 * EMBED_DIM] = t2_ref[...].T


_transpose = pl.pallas_call(
    _tr_body,
    out_shape=jax.ShapeDtypeStruct((VOC0, 2 * EMBED_DIM), jnp.float32),
    grid=(NTB,),
    in_specs=[
        pl.BlockSpec((EMBED_DIM, _LB), lambda i: (0, i)),
        pl.BlockSpec((EMBED_DIM, _LB),
                     lambda i: (0, jnp.minimum(NTB + i, _NIN - 1))),
    ],
    out_specs=pl.BlockSpec((_LB, 2 * EMBED_DIM), lambda i: (i, 0)),
)


CHR = 320
NCH = VOCAB // CHR
CPT = -(-NCH // NW)


def _repack_body(table_hbm, out_hbm, vin0, vin1, vout0, vout1,
                 semi0, semi1, semo0, semo1):
  wid = lax.axis_index("s") * NC + lax.axis_index("c")
  semi = {id(vin0): semi0, id(vin1): semi1}
  semo = {id(vout0): semo0, id(vout1): semo1}

  def _in_copy(g, vin):
    return pltpu.make_async_copy(table_hbm.at[pl.ds(g * CHR, CHR)], vin,
                                 semi[id(vin)])

  def _out_copy(g, vout):
    return pltpu.make_async_copy(vout, out_hbm.at[pl.ds(g * (CHR // 2),
                                                        CHR // 2)],
                                 semo[id(vout)])

  def repack(vin, vout):
    def pair_body(p, carry):
      for rr in range(2):
        for c in range(NLANE):
          vout[p, pl.ds(rr * EMBED_DIM + c * 16, 16)] = (
              vin[p * 2 + rr, pl.ds(c * 16, 16)])
      return carry
    lax.fori_loop(0, CHR // 2, pair_body, 0)

  def chunk_of(t):
    return t * NW + wid

  @pl.when(chunk_of(0) < NCH)
  def _():
    _in_copy(chunk_of(0), vin0).start()

  def body2(i, carry):
    t = i * 2
    for tt, vin_a, vout_a, vin_b in ((t, vin0, vout0, vin1),
                                     (t + 1, vin1, vout1, vin0)):
      g = chunk_of(tt)

      @pl.when(chunk_of(tt + 1) < NCH)
      def _():
        _in_copy(chunk_of(tt + 1), vin_b).start()

      @pl.when(g < NCH)
      def _():
        _in_copy(g, vin_a).wait()
        @pl.when(tt >= 2)
        def _():
          _out_copy(chunk_of(tt - 2), vout_a).wait()
        repack(vin_a, vout_a)
        _out_copy(g, vout_a).start()
    return carry

  lax.fori_loop(0, CPT // 2, body2, 0)

  @pl.when(chunk_of(CPT - 2) < NCH)
  def _():
    _out_copy(chunk_of(CPT - 2), vout0).wait()

  @pl.when(chunk_of(CPT - 1) < NCH)
  def _():
    _out_copy(chunk_of(CPT - 1), vout1).wait()


_repack = functools.partial(
    pl.kernel,
    out_type=jax.ShapeDtypeStruct((VOCAB // 2, ROWW), jnp.float32),
    mesh=plsc.VectorSubcoreMesh(core_axis_name="c", subcore_axis_name="s",
                                num_cores=NC, num_subcores=NS),
    scratch_types=[
        pltpu.VMEM((CHR, EMBED_DIM), jnp.float32),
        pltpu.VMEM((CHR, EMBED_DIM), jnp.float32),
        pltpu.VMEM((CHR // 2, ROWW), jnp.float32),
        pltpu.VMEM((CHR // 2, ROWW), jnp.float32),
        pltpu.SemaphoreType.DMA,
        pltpu.SemaphoreType.DMA,
        pltpu.SemaphoreType.DMA,
        pltpu.SemaphoreType.DMA,
    ],
)(_repack_body)


def _sc_pool_body(ids_hbm, table_hbm, out_hbm, idx_v, rows0, rows1,
                  pidx0, pidx1, pooled_v, sem0, sem1):
  wid = lax.axis_index("s") * NC + lax.axis_index("c")
  base = wid * BPW

  pltpu.sync_copy(ids_hbm.at[pl.ds(base * SEQP, BPW * SEQP)], idx_v)

  zeros16 = jnp.zeros((16,), jnp.float32)
  for buf in (rows0, rows1):
    for rz in range(SEQ, SEQB):
      for c in range(ROWW // 16):
        buf[rz, pl.ds(c * 16, 16)] = zeros16

  def _row_copies(r, buf, pidx, sem):
    return (
        pltpu.make_async_copy(table_hbm.at[pidx.at[pl.ds(0, SPLIT0)]],
                              buf.at[pl.ds(0, SPLIT0)], sem),
        pltpu.make_async_copy(table_hbm.at[pidx.at[pl.ds(SPLIT0, SPLIT1)]],
                              buf.at[pl.ds(SPLIT0, SPLIT1)], sem),
    )

  def start_row(r, buf, pidx, sem):
    off = r * SEQP
    for j in range(NBLK):
      iv = idx_v[pl.ds(off + j * 16, 16)]
      pidx[pl.ds(j * 16, 16)] = iv - jnp.where(iv >= VOC0, VOC0, 0)
    for cp in _row_copies(r, buf, pidx, sem):
      cp.start()

  def wait_buf(r, buf, pidx, sem):
    for cp in _row_copies(r, buf, pidx, sem):
      cp.wait()

  def reduce_row(buf, r):
    init = tuple(jnp.full((16,), -jnp.inf, dtype=jnp.float32)
                 for _ in range(NLANE))
    def blk_body(j, accs):
      accs = list(accs)
      l0 = j * 16
      iv = idx_v[pl.ds(r * SEQP + l0, 16)]
      mv = jnp.where(iv == PAD_IDX, jnp.float32(0), jnp.float32(1))
      hv = jnp.where(iv >= VOC0, EMBED_DIM, 0)
      for u in range(16):
        m = mv[u]
        h = hv[u]
        for c in range(NLANE):
          v = buf[l0 + u, pl.ds(h + c * 16, 16)]
          accs[c] = jnp.maximum(accs[c], v * m)
      return tuple(accs)
    accs = lax.fori_loop(0, NBLK, blk_body, init)
    for c in range(NLANE):
      pooled_v[r, pl.ds(c * 16, 16)] = accs[c]

  start_row(0, rows0, pidx0, sem0)

  def body2(i, carry):
    r = i * 2
    start_row(r + 1, rows1, pidx1, sem1)
    wait_buf(r, rows0, pidx0, sem0)
    reduce_row(rows0, r)

    @pl.when(r + 2 < BPW)
    def _():
      start_row(r + 2, rows0, pidx0, sem0)

    wait_buf(r + 1, rows1, pidx1, sem1)
    reduce_row(rows1, r + 1)
    return carry

  lax.fori_loop(0, BPW // 2, body2, 0)
  pltpu.sync_copy(pooled_v, out_hbm.at[pl.ds(base, BPW)])


_sc_pool = functools.partial(
    pl.kernel,
    out_type=jax.ShapeDtypeStruct((BATCH, EMBED_DIM), jnp.float32),
    mesh=plsc.VectorSubcoreMesh(core_axis_name="c", subcore_axis_name="s",
                                num_cores=NC, num_subcores=NS),
    scratch_types=[
        pltpu.VMEM((BPW * SEQP,), jnp.int32),
        pltpu.VMEM((SEQB, ROWW), jnp.float32),
        pltpu.VMEM((SEQB, ROWW), jnp.float32),
        pltpu.VMEM((SEQB,), jnp.int32),
        pltpu.VMEM((SEQB,), jnp.int32),
        pltpu.VMEM((BPW, EMBED_DIM), jnp.float32),
        pltpu.SemaphoreType.DMA,
        pltpu.SemaphoreType.DMA,
    ],
)(_sc_pool_body)


def _head_body(p_ref, wt_ref, b_ref, o_ref):
  logits = jnp.dot(p_ref[...], wt_ref[...],
                   preferred_element_type=jnp.float32) + b_ref[...]
  mx = jnp.max(logits, axis=1, keepdims=True)
  sh = logits - mx
  lse = jnp.log(jnp.sum(jnp.exp(sh), axis=1, keepdims=True))
  o_ref[...] = sh - lse


_BB = 1024

_head = pl.pallas_call(
    _head_body,
    out_shape=jax.ShapeDtypeStruct((BATCH, NUM_LABELS), jnp.float32),
    grid=(BATCH // _BB,),
    in_specs=[
        pl.BlockSpec((_BB, EMBED_DIM), lambda i: (i, 0)),
        pl.BlockSpec((EMBED_DIM, NUM_LABELS), lambda i: (0, 0)),
        pl.BlockSpec((1, NUM_LABELS), lambda i: (0, 0)),
    ],
    out_specs=pl.BlockSpec((_BB, NUM_LABELS), lambda i: (i, 0)),
)


def kernel(text, sequence_lens, table, W, b):
  del sequence_lens
  ids = jnp.pad(text.astype(jnp.int32), ((0, 0), (0, SEQP - SEQ)),
                constant_values=PAD_IDX).reshape(-1)
  tt = table.T
  table2 = _transpose(tt, tt)
  pooled = _sc_pool(ids, table2)
  return _head(pooled, W.T, b.reshape(1, NUM_LABELS))


# speedup vs baseline: 2.3207x; 1.0063x over previous
import functools

import jax
import jax.numpy as jnp
from jax import lax
from jax.experimental import pallas as pl
from jax.experimental.pallas import tpu as pltpu
from jax.experimental.pallas import tpu_sc as plsc

VOCAB = 1000000
EMBED_DIM = 64
NUM_LABELS = 50
BATCH = 4096
SEQ = 200
PAD_IDX = 3000

NC = 2
NS = 16
NW = NC * NS
BPW = BATCH // NW
SEQP = 256
SEQB = 208
NBLK = SEQB // 16
SPLIT0 = 104
SPLIT1 = SEQ - SPLIT0
NLANE = EMBED_DIM // 16
ROWW = 2 * EMBED_DIM


_LB = 16384
NTB = 31
VOC0 = NTB * _LB
_NIN = -(-VOCAB // _LB)


def _tr_body(t1_ref, t2_ref, o_ref):
  o_ref[:, 0:EMBED_DIM] = t1_ref[...].T
  o_ref[:, EMBED_DIM:2 * EMBED_DIM] = t2_ref[...].T


_transpose = pl.pallas_call(
    _tr_body,
    out_shape=jax.ShapeDtypeStruct((VOC0, 2 * EMBED_DIM), jnp.float32),
    grid=(NTB,),
    in_specs=[
        pl.BlockSpec((EMBED_DIM, _LB), lambda i: (0, i)),
        pl.BlockSpec((EMBED_DIM, _LB),
                     lambda i: (0, jnp.minimum(NTB + i, _NIN - 1))),
    ],
    out_specs=pl.BlockSpec((_LB, 2 * EMBED_DIM), lambda i: (i, 0)),
)


def _sc_pool_body(ids_hbm, table_hbm, out_hbm, idx_v, rows0, rows1,
                  pidx0, pidx1, pooled_v, sem0, sem1):
  wid = lax.axis_index("s") * NC + lax.axis_index("c")
  base = wid * BPW

  pltpu.sync_copy(ids_hbm.at[pl.ds(base * SEQP, BPW * SEQP)], idx_v)

  zeros16 = jnp.zeros((16,), jnp.float32)
  for buf in (rows0, rows1):
    for rz in range(SEQ, SEQB):
      for c in range(ROWW // 16):
        buf[rz, pl.ds(c * 16, 16)] = zeros16

  def _row_copies(r, buf, pidx, sem):
    return (
        pltpu.make_async_copy(table_hbm.at[pidx.at[pl.ds(0, SPLIT0)]],
                              buf.at[pl.ds(0, SPLIT0)], sem),
        pltpu.make_async_copy(table_hbm.at[pidx.at[pl.ds(SPLIT0, SPLIT1)]],
                              buf.at[pl.ds(SPLIT0, SPLIT1)], sem),
    )

  def start_row(r, buf, pidx, sem):
    off = r * SEQP
    for j in range(NBLK):
      iv = idx_v[pl.ds(off + j * 16, 16)]
      pidx[pl.ds(j * 16, 16)] = iv - jnp.where(iv >= VOC0, VOC0, 0)
    for cp in _row_copies(r, buf, pidx, sem):
      cp.start()

  def wait_buf(r, buf, pidx, sem):
    for cp in _row_copies(r, buf, pidx, sem):
      cp.wait()

  def reduce_row(buf, r):
    init = tuple(jnp.full((16,), -jnp.inf, dtype=jnp.float32)
                 for _ in range(NLANE))
    def blk_body(j, accs):
      accs = list(accs)
      l0 = j * 16
      iv = idx_v[pl.ds(r * SEQP + l0, 16)]
      mv = jnp.where(iv == PAD_IDX, jnp.float32(0), jnp.float32(1))
      hv = jnp.where(iv >= VOC0, EMBED_DIM, 0)
      for u in range(16):
        m = mv[u]
        h = hv[u]
        for c in range(NLANE):
          v = buf[l0 + u, pl.ds(h + c * 16, 16)]
          accs[c] = jnp.maximum(accs[c], v * m)
      return tuple(accs)
    accs = lax.fori_loop(0, NBLK, blk_body, init)
    for c in range(NLANE):
      pooled_v[r, pl.ds(c * 16, 16)] = accs[c]

  start_row(0, rows0, pidx0, sem0)

  def body2(i, carry):
    r = i * 2
    start_row(r + 1, rows1, pidx1, sem1)
    wait_buf(r, rows0, pidx0, sem0)
    reduce_row(rows0, r)

    @pl.when(r + 2 < BPW)
    def _():
      start_row(r + 2, rows0, pidx0, sem0)

    wait_buf(r + 1, rows1, pidx1, sem1)
    reduce_row(rows1, r + 1)
    return carry

  lax.fori_loop(0, BPW // 2, body2, 0)
  pltpu.sync_copy(pooled_v, out_hbm.at[pl.ds(base, BPW)])


_sc_pool = functools.partial(
    pl.kernel,
    out_type=jax.ShapeDtypeStruct((BATCH, EMBED_DIM), jnp.float32),
    mesh=plsc.VectorSubcoreMesh(core_axis_name="c", subcore_axis_name="s",
                                num_cores=NC, num_subcores=NS),
    scratch_types=[
        pltpu.VMEM((BPW * SEQP,), jnp.int32),
        pltpu.VMEM((SEQB, ROWW), jnp.float32),
        pltpu.VMEM((SEQB, ROWW), jnp.float32),
        pltpu.VMEM((SEQB,), jnp.int32),
        pltpu.VMEM((SEQB,), jnp.int32),
        pltpu.VMEM((BPW, EMBED_DIM), jnp.float32),
        pltpu.SemaphoreType.DMA,
        pltpu.SemaphoreType.DMA,
    ],
)(_sc_pool_body)


def _head_body(p_ref, wt_ref, b_ref, o_ref):
  logits = jnp.dot(p_ref[...], wt_ref[...],
                   preferred_element_type=jnp.float32) + b_ref[...]
  mx = jnp.max(logits, axis=1, keepdims=True)
  sh = logits - mx
  lse = jnp.log(jnp.sum(jnp.exp(sh), axis=1, keepdims=True))
  o_ref[...] = sh - lse


_BB = 1024

_head = pl.pallas_call(
    _head_body,
    out_shape=jax.ShapeDtypeStruct((BATCH, NUM_LABELS), jnp.float32),
    grid=(BATCH // _BB,),
    in_specs=[
        pl.BlockSpec((_BB, EMBED_DIM), lambda i: (i, 0)),
        pl.BlockSpec((EMBED_DIM, NUM_LABELS), lambda i: (0, 0)),
        pl.BlockSpec((1, NUM_LABELS), lambda i: (0, 0)),
    ],
    out_specs=pl.BlockSpec((_BB, NUM_LABELS), lambda i: (i, 0)),
)


def kernel(text, sequence_lens, table, W, b):
  del sequence_lens
  ids = jnp.pad(text.astype(jnp.int32), ((0, 0), (0, SEQP - SEQ)),
                constant_values=PAD_IDX).reshape(-1)
  tt = table.T
  table2 = _transpose(tt, tt)
  pooled = _sc_pool(ids, table2)
  return _head(pooled, W.T, b.reshape(1, NUM_LABELS))
